# baseline (speedup 1.0000x reference)
import jax
import jax.numpy as jnp
from jax.experimental import pallas as pl
from jax.experimental.pallas import tpu as pltpu

_B, _C_IN, _C_OUT, _H, _W = 8, 16, 16, 512, 512
_TH = 256


def _spconv_kern(x_ref, wcat_ref, o_ref):
    xb = x_ref[0].reshape(_C_IN, _TH * _W)
    ones8 = jnp.ones((8, _C_IN), dtype=jnp.bfloat16)
    a = jnp.abs(xb).astype(jnp.bfloat16)
    s = jax.lax.dot_general(
        ones8, a, (((1,), (0,)), ((), ())), preferred_element_type=jnp.float32
    )
    maskf = jnp.where(s > 0, 1.0, 0.0).astype(jnp.float32)
    aug = jnp.concatenate([xb, maskf], axis=0)
    out = jax.lax.dot_general(
        wcat_ref[...], aug, (((1,), (0,)), ((), ())),
        preferred_element_type=jnp.float32,
    )
    o_ref[0] = out.reshape(_C_OUT, _TH, _W)


def kernel(x, W, b):
    wcat = jnp.concatenate(
        [W, b.reshape(_C_OUT, 1), jnp.zeros((_C_OUT, 7), jnp.float32)], axis=1
    )
    out = pl.pallas_call(
        _spconv_kern,
        grid=(_B, _H // _TH),
        in_specs=[
            pl.BlockSpec((1, _C_IN, _TH, _W), lambda i, j: (i, 0, j, 0)),
            pl.BlockSpec((_C_OUT, _C_IN + 8), lambda i, j: (0, 0)),
        ],
        out_specs=pl.BlockSpec((1, _C_OUT, _TH, _W), lambda i, j: (i, 0, j, 0)),
        out_shape=jax.ShapeDtypeStruct((_B, _C_OUT, _H, _W), jnp.float32),
        compiler_params=pltpu.CompilerParams(
            dimension_semantics=("parallel", "parallel")),
    )(x, wcat)
    return out
